# baseline (device time: 22902 ns/iter reference)
import jax
import jax.numpy as jnp
from jax import lax
from jax.experimental import pallas as pl
from jax.experimental.pallas import tpu as pltpu

N_DEV = 32
NQ = 8
NR = 4
BLK = 32
K = 1024
N_OUT = 1024

_P2_PROC = (1, 7, 2, 6, 3, 5, 4)
_P2_SEND = (4, 3, 5, 2, 6, 1, 7)


def kernel(x, w_mat):
    def body(
        x_ref,
        w_ref,
        out_ref,
        xs_ref,
        rbuf1,
        rs_ref,
        rbuf2,
        p1_send,
        p1_recv,
        p2_send,
        p2_recv,
        p2_bar,
        loc_sems,
    ):
        me = lax.axis_index("i")
        q_me = lax.div(me, NR)
        r_me = lax.rem(me, NR)

        def p1_desc(j):
            rp = lax.rem(r_me + j, NR)
            return pltpu.make_async_remote_copy(
                src_ref=xs_ref.at[rp],
                dst_ref=rbuf1.at[r_me],
                send_sem=p1_send.at[j],
                recv_sem=p1_recv.at[j],
                device_id=(q_me * NR + rp,),
                device_id_type=pl.DeviceIdType.MESH,
            )

        def p2_desc(j):
            kt = lax.rem(q_me + j, NQ)
            return pltpu.make_async_remote_copy(
                src_ref=rs_ref.at[kt],
                dst_ref=rbuf2.at[q_me],
                send_sem=p2_send.at[j],
                recv_sem=p2_recv.at[j],
                device_id=(kt * NR + r_me,),
                device_id_type=pl.DeviceIdType.MESH,
            )

        def chunk_update(acc, Q):
            ck = jnp.transpose(rbuf2[Q], (1, 0, 2)).reshape(BLK, NR * BLK)
            return acc + jnp.dot(
                ck,
                w_ref[pl.ds(Q * NR * BLK, NR * BLK), :],
                preferred_element_type=jnp.float32,
            )

        barrier_sem = pltpu.get_barrier_semaphore()
        for j in range(1, NR):
            pl.semaphore_signal(
                barrier_sem,
                inc=1,
                device_id=(q_me * NR + lax.rem(r_me + j, NR),),
                device_id_type=pl.DeviceIdType.MESH,
            )
        xs_ref[...] = jnp.swapaxes(x_ref[...], 0, 1)
        pl.semaphore_wait(barrier_sem, NR - 1)

        for j in range(1, NR):
            p1_desc(j).start()
        copy1 = pltpu.make_async_copy(
            xs_ref.at[r_me], rbuf1.at[r_me], loc_sems.at[0]
        )
        copy1.start()
        for j in range(1, NQ):
            pl.semaphore_signal(
                p2_bar,
                inc=1,
                device_id=(lax.rem(q_me + j, NQ) * NR + r_me,),
                device_id_type=pl.DeviceIdType.MESH,
            )

        for j in range(1, NR):
            p1_desc(j).wait_recv()
        copy1.wait()

        rs_ref[...] = jnp.swapaxes(rbuf1[...], 0, 1)
        copy2 = pltpu.make_async_copy(
            rs_ref.at[q_me], rbuf2.at[q_me], loc_sems.at[1]
        )
        copy2.start()

        pl.semaphore_wait(p2_bar, NQ - 1)
        for j in _P2_SEND:
            p2_desc(j).start()
        for j in range(1, NR):
            p1_desc(j).wait_send()

        copy2.wait()
        acc = chunk_update(jnp.zeros((BLK, N_OUT), jnp.float32), q_me)

        for j in _P2_PROC:
            p2_desc(j).wait_recv()
            acc = chunk_update(acc, lax.rem(q_me + NQ - j, NQ))
            p2_desc(j).wait_send()

        out_ref[...] = jnp.maximum(acc, 0.0)

    return pl.pallas_call(
        body,
        out_shape=jax.ShapeDtypeStruct((BLK, N_OUT), jnp.float32),
        in_specs=[
            pl.BlockSpec(memory_space=pltpu.VMEM),
            pl.BlockSpec(memory_space=pltpu.VMEM),
        ],
        out_specs=pl.BlockSpec(memory_space=pltpu.VMEM),
        scratch_shapes=[
            pltpu.VMEM((NR, NQ, BLK, BLK), jnp.float32),
            pltpu.VMEM((NR, NQ, BLK, BLK), jnp.float32),
            pltpu.VMEM((NQ, NR, BLK, BLK), jnp.float32),
            pltpu.VMEM((NQ, NR, BLK, BLK), jnp.float32),
            pltpu.SemaphoreType.DMA((NR,)),
            pltpu.SemaphoreType.DMA((NR,)),
            pltpu.SemaphoreType.DMA((NQ,)),
            pltpu.SemaphoreType.DMA((NQ,)),
            pltpu.SemaphoreType.REGULAR,
            pltpu.SemaphoreType.DMA((2,)),
        ],
        compiler_params=pltpu.CompilerParams(collective_id=0),
    )(x.reshape(NQ, NR, BLK, BLK), w_mat)


# device time: 21804 ns/iter; 1.0504x vs baseline; 1.0504x over previous
import jax
import jax.numpy as jnp
from jax import lax
from jax.experimental import pallas as pl
from jax.experimental.pallas import tpu as pltpu

N_DEV = 32
NQ = 8
NR = 4
BLK = 32
K = 1024
N_OUT = 1024

_SEND_ORDER = sorted(range(1, N_DEV), key=lambda o: -min(o, N_DEV - o))


def kernel(x, w_mat):
    def body(
        x_ref,
        w_hbm,
        out_ref,
        w_ref,
        x3_ref,
        send_sems,
        recv_sems,
        up_sem,
        down_sem,
        w_sem,
    ):
        me = lax.axis_index("i")
        q_me = lax.div(me, NR)
        r_me = lax.rem(me, NR)
        leader = q_me * NR

        wcopy = pltpu.make_async_copy(w_hbm, w_ref, w_sem)
        wcopy.start()

        x3_ref[me] = x_ref[pl.ds(me * BLK, BLK), :]

        @pl.when(r_me != 0)
        def _():
            pl.semaphore_signal(
                up_sem,
                inc=1,
                device_id=(leader,),
                device_id_type=pl.DeviceIdType.MESH,
            )
            pl.semaphore_wait(down_sem, 1)

        xs_sem = pltpu.get_barrier_semaphore()

        @pl.when(r_me == 0)
        def _():
            pl.semaphore_wait(up_sem, NR - 1)
            for j in range(1, NQ):
                pl.semaphore_signal(
                    xs_sem,
                    inc=1,
                    device_id=(lax.rem(q_me + j, NQ) * NR,),
                    device_id_type=pl.DeviceIdType.MESH,
                )
            pl.semaphore_wait(xs_sem, NQ - 1)
            for j in range(1, NR):
                pl.semaphore_signal(
                    down_sem,
                    inc=1,
                    device_id=(leader + j,),
                    device_id_type=pl.DeviceIdType.MESH,
                )

        def desc(off):
            dst = lax.rem(me + off, N_DEV)
            return pltpu.make_async_remote_copy(
                src_ref=x_ref.at[pl.ds(dst * BLK, BLK), :],
                dst_ref=x3_ref.at[me],
                send_sem=send_sems.at[off],
                recv_sem=recv_sems.at[off],
                device_id=(dst,),
                device_id_type=pl.DeviceIdType.MESH,
            )

        def rdesc(off):
            src = lax.rem(me + N_DEV - off, N_DEV)
            return pltpu.make_async_remote_copy(
                src_ref=x_ref.at[pl.ds(src * BLK, BLK), :],
                dst_ref=x3_ref.at[src],
                send_sem=send_sems.at[off],
                recv_sem=recv_sems.at[off],
                device_id=(src,),
                device_id_type=pl.DeviceIdType.MESH,
            )

        for off in _SEND_ORDER:
            desc(off).start()
        for off in range(1, N_DEV):
            rdesc(off).wait_recv()
        for off in range(1, N_DEV):
            desc(off).wait_send()

        xr = jnp.transpose(x3_ref[...], (1, 0, 2)).reshape(BLK, K)
        wcopy.wait()
        out_ref[...] = jnp.maximum(
            jnp.dot(xr, w_ref[...], preferred_element_type=jnp.float32), 0.0
        )

    return pl.pallas_call(
        body,
        out_shape=jax.ShapeDtypeStruct((BLK, N_OUT), jnp.float32),
        in_specs=[
            pl.BlockSpec(memory_space=pltpu.VMEM),
            pl.BlockSpec(memory_space=pltpu.MemorySpace.HBM),
        ],
        out_specs=pl.BlockSpec(memory_space=pltpu.VMEM),
        scratch_shapes=[
            pltpu.VMEM((K, N_OUT), jnp.float32),
            pltpu.VMEM((N_DEV, BLK, BLK), jnp.float32),
            pltpu.SemaphoreType.DMA((N_DEV,)),
            pltpu.SemaphoreType.DMA((N_DEV,)),
            pltpu.SemaphoreType.REGULAR,
            pltpu.SemaphoreType.REGULAR,
            pltpu.SemaphoreType.DMA,
        ],
        compiler_params=pltpu.CompilerParams(collective_id=0),
    )(x, w_mat)


# device time: 19848 ns/iter; 1.1539x vs baseline; 1.0985x over previous
import jax
import jax.numpy as jnp
from jax import lax
from jax.experimental import pallas as pl
from jax.experimental.pallas import tpu as pltpu

N_DEV = 32
BLK = 32
K = 1024
N_OUT = 1024

_SEND_ORDER = sorted(range(1, N_DEV), key=lambda o: -min(o, N_DEV - o))


def kernel(x, w_mat):
    def body(
        x_ref,
        w_hbm,
        out_ref,
        w_ref,
        x3_ref,
        send_sems,
        recv_sems,
        w_sem,
    ):
        me = lax.axis_index("i")

        wcopy = pltpu.make_async_copy(w_hbm, w_ref, w_sem)
        wcopy.start()

        x3_ref[me] = x_ref[pl.ds(me * BLK, BLK), :]

        barrier_sem = pltpu.get_barrier_semaphore()
        for off in range(1, N_DEV):
            pl.semaphore_signal(
                barrier_sem,
                inc=1,
                device_id=(lax.rem(me + off, N_DEV),),
                device_id_type=pl.DeviceIdType.MESH,
            )
        pl.semaphore_wait(barrier_sem, N_DEV - 1)

        def desc(off):
            dst = lax.rem(me + off, N_DEV)
            return pltpu.make_async_remote_copy(
                src_ref=x_ref.at[pl.ds(dst * BLK, BLK), :],
                dst_ref=x3_ref.at[me],
                send_sem=send_sems.at[off],
                recv_sem=recv_sems.at[off],
                device_id=(dst,),
                device_id_type=pl.DeviceIdType.MESH,
            )

        def rdesc(off):
            src = lax.rem(me + N_DEV - off, N_DEV)
            return pltpu.make_async_remote_copy(
                src_ref=x_ref.at[pl.ds(src * BLK, BLK), :],
                dst_ref=x3_ref.at[src],
                send_sem=send_sems.at[off],
                recv_sem=recv_sems.at[off],
                device_id=(src,),
                device_id_type=pl.DeviceIdType.MESH,
            )

        for off in _SEND_ORDER:
            desc(off).start()
        for off in range(1, N_DEV):
            rdesc(off).wait_recv()
        for off in range(1, N_DEV):
            desc(off).wait_send()

        xr = jnp.transpose(x3_ref[...], (1, 0, 2)).reshape(BLK, K)
        wcopy.wait()
        out_ref[...] = jnp.maximum(
            jnp.dot(xr, w_ref[...], preferred_element_type=jnp.float32), 0.0
        )

    return pl.pallas_call(
        body,
        out_shape=jax.ShapeDtypeStruct((BLK, N_OUT), jnp.float32),
        in_specs=[
            pl.BlockSpec(memory_space=pltpu.VMEM),
            pl.BlockSpec(memory_space=pltpu.MemorySpace.HBM),
        ],
        out_specs=pl.BlockSpec(memory_space=pltpu.VMEM),
        scratch_shapes=[
            pltpu.VMEM((K, N_OUT), jnp.float32),
            pltpu.VMEM((N_DEV, BLK, BLK), jnp.float32),
            pltpu.SemaphoreType.DMA((N_DEV,)),
            pltpu.SemaphoreType.DMA((N_DEV,)),
            pltpu.SemaphoreType.DMA,
        ],
        compiler_params=pltpu.CompilerParams(collective_id=0),
    )(x, w_mat)
